# baseline (speedup 1.0000x reference)
import functools

import jax
import jax.numpy as jnp
from jax import lax
from jax.experimental import pallas as pl
from jax.experimental.pallas import tpu as pltpu
from jax.experimental.pallas import tpu_sc as plsc

_N = 10000
_E = 320000
_D_IN = 128
_D_H = 64

_NC = 2
_NS = 16
_CHUNK = 128
_CPT = 160
_EP = _NS * _CPT * _CHUNK
_R = 10112
_RPT = _R // _NS
_DEPTH = 4


def _sc_mesh():
    return plsc.VectorSubcoreMesh(core_axis_name="c", subcore_axis_name="s",
                                  num_cores=_NC, num_subcores=_NS)


def _sc_agg_body(with_count, *refs):
    if with_count:
        (tbl, src3, dst3, zrows, zrows8, ones, aggp, cntp,
         agg_sh, cnt_sh, sidx2, didx2, ones_v, csem,
         *bufs) = refs
    else:
        (tbl, src3, dst3, zrows, aggp,
         agg_sh, sidx2, didx2, *bufs) = refs
    rows = bufs[:_DEPTH]
    gsems = bufs[_DEPTH:2 * _DEPTH]
    ssems = bufs[2 * _DEPTH:]
    c = lax.axis_index("c")
    s = lax.axis_index("s")
    rbase = s * _RPT

    @pl.when(c == 0)
    def _gather_core():
        pltpu.sync_copy(zrows, agg_sh.at[pl.ds(rbase, _RPT)])
        pltpu.sync_copy(src3.at[s], sidx2)
        pltpu.sync_copy(dst3.at[s], didx2)
        plsc.subcore_barrier()

        for m in range(_DEPTH - 1):
            pltpu.async_copy(tbl.at[sidx2.at[m]], rows[m], gsems[m])

        def step(j, carry):
            for m in range(_DEPTH):
                @pl.when(j % _DEPTH == m)
                def _(m=m):
                    pltpu.make_async_copy(tbl.at[sidx2.at[j]], rows[m],
                                          gsems[m]).wait()
                    pltpu.async_copy(rows[m], agg_sh.at[didx2.at[j]],
                                     ssems[m], add=True)
                    nxt = j + _DEPTH - 1
                    b = (m + _DEPTH - 1) % _DEPTH

                    @pl.when(nxt < _CPT)
                    def _():
                        @pl.when(j >= 1)
                        def _():
                            pltpu.make_async_copy(rows[b],
                                                  agg_sh.at[didx2.at[0]],
                                                  ssems[b]).wait()

                        pltpu.async_copy(tbl.at[sidx2.at[nxt]], rows[b],
                                         gsems[b])
            return carry

        lax.fori_loop(0, _CPT, step, 0)
        for m in range(_DEPTH):
            pltpu.make_async_copy(rows[m], agg_sh.at[didx2.at[0]],
                                  ssems[m]).wait()
        plsc.subcore_barrier()
        pltpu.sync_copy(agg_sh.at[pl.ds(rbase, _RPT)],
                        aggp.at[pl.ds(rbase, _RPT)])

    if with_count:
        @pl.when(c == 1)
        def _count_core():
            pltpu.sync_copy(zrows8, cnt_sh.at[pl.ds(rbase, _RPT)])
            pltpu.sync_copy(ones, ones_v)
            pltpu.sync_copy(dst3.at[s], didx2)
            plsc.subcore_barrier()

            def cstep(j, carry):
                pltpu.async_copy(ones_v, cnt_sh.at[didx2.at[j]], csem,
                                 add=True)

                @pl.when(j >= _DEPTH)
                def _():
                    pltpu.make_async_copy(ones_v, cnt_sh.at[didx2.at[0]],
                                          csem).wait()
                return carry

            lax.fori_loop(0, _CPT, cstep, 0)
            for _ in range(_DEPTH):
                pltpu.make_async_copy(ones_v, cnt_sh.at[didx2.at[0]],
                                      csem).wait()
            plsc.subcore_barrier()
            pltpu.sync_copy(cnt_sh.at[pl.ds(rbase, _RPT)],
                            cntp.at[pl.ds(rbase, _RPT)])


def _sc_aggregate1(p1, src3, dst3, zrows, zrows8, ones):
    fn = pl.kernel(
        functools.partial(_sc_agg_body, True),
        out_type=(
            jax.ShapeDtypeStruct((_R, _D_H), jnp.float32),
            jax.ShapeDtypeStruct((_R, 8), jnp.float32),
        ),
        mesh=_sc_mesh(),
        compiler_params=pltpu.CompilerParams(use_tc_tiling_on_sc=False),
        scratch_types=[
            pltpu.VMEM_SHARED((_R, _D_H), jnp.float32),
            pltpu.VMEM_SHARED((_R, 8), jnp.float32),
            pltpu.VMEM((_CPT, _CHUNK), jnp.int32),
            pltpu.VMEM((_CPT, _CHUNK), jnp.int32),
            pltpu.VMEM((_CHUNK, 8), jnp.float32),
            pltpu.SemaphoreType.DMA,
        ] + [pltpu.VMEM((_CHUNK, _D_H), jnp.float32)] * _DEPTH
          + [pltpu.SemaphoreType.DMA] * (2 * _DEPTH),
    )
    return fn(p1, src3, dst3, zrows, zrows8, ones)


def _sc_aggregate2(p2, src3, dst3, zrows8):
    fn = pl.kernel(
        functools.partial(_sc_agg_body, False),
        out_type=jax.ShapeDtypeStruct((_R, 8), jnp.float32),
        mesh=_sc_mesh(),
        compiler_params=pltpu.CompilerParams(use_tc_tiling_on_sc=False),
        scratch_types=[
            pltpu.VMEM_SHARED((_R, 8), jnp.float32),
            pltpu.VMEM((_CPT, _CHUNK), jnp.int32),
            pltpu.VMEM((_CPT, _CHUNK), jnp.int32),
        ] + [pltpu.VMEM((_CHUNK, 8), jnp.float32)] * _DEPTH
          + [pltpu.SemaphoreType.DMA] * (2 * _DEPTH),
    )
    return fn(p2, src3, dst3, zrows8)


def _tc_proj1_body(x_ref, wl_ref, wr_ref, b_ref, p_ref, r_ref):
    x = x_ref[...]
    p_ref[...] = jnp.dot(x, wl_ref[...], preferred_element_type=jnp.float32)
    r_ref[...] = (jnp.dot(x, wr_ref[...], preferred_element_type=jnp.float32)
                  + b_ref[...])


def _tc_mid_body(a_ref, c_ref, r1_ref, wl_ref, wr_ref,
                 b_ref, p2_ref, r2_ref):
    cnt = jnp.maximum(c_ref[...], 1.0)
    h = jnp.maximum(a_ref[...] / cnt + r1_ref[...], 0.0)
    p2_ref[...] = jnp.dot(h, wl_ref[...], preferred_element_type=jnp.float32)
    r2_ref[...] = (jnp.dot(h, wr_ref[...], preferred_element_type=jnp.float32)
                   + b_ref[...])


def _tc_out_body(a_ref, c_ref, r2_ref, o_ref):
    cnt = jnp.maximum(c_ref[...], 1.0)
    o_ref[...] = a_ref[...] / cnt + r2_ref[...]


def kernel(x, edge_index, W1_l, W1_r, b1, W2_l, W2_r, b2):
    src = edge_index[0]
    dst = edge_index[1]
    pad = _EP - _E
    src3 = jnp.concatenate([src, jnp.zeros((pad,), jnp.int32)]
                           ).reshape(_NS, _CPT, _CHUNK)
    dst3 = jnp.concatenate([dst, jnp.full((pad,), _N, jnp.int32)]
                           ).reshape(_NS, _CPT, _CHUNK)
    zrows = jnp.zeros((_RPT, _D_H), jnp.float32)
    zrows8 = jnp.zeros((_RPT, 8), jnp.float32)
    ones = jnp.ones((_CHUNK, 8), jnp.float32)

    p1, r1 = pl.pallas_call(
        _tc_proj1_body,
        out_shape=(
            jax.ShapeDtypeStruct((_N, _D_H), jnp.float32),
            jax.ShapeDtypeStruct((_N, _D_H), jnp.float32),
        ),
    )(x, W1_l.T, W1_r.T, b1[None, :])

    aggp, cntp = _sc_aggregate1(p1, src3, dst3, zrows, zrows8, ones)

    w2l8 = jnp.zeros((_D_H, 8), jnp.float32).at[:, :2].set(W2_l.T)
    w2r8 = jnp.zeros((_D_H, 8), jnp.float32).at[:, :2].set(W2_r.T)
    b2_8 = jnp.zeros((1, 8), jnp.float32).at[0, :2].set(b2)
    p2, r2 = pl.pallas_call(
        _tc_mid_body,
        out_shape=(
            jax.ShapeDtypeStruct((_N, 8), jnp.float32),
            jax.ShapeDtypeStruct((_N, 8), jnp.float32),
        ),
    )(aggp[:_N], cntp[:_N, :1], r1, w2l8, w2r8, b2_8)

    agg2p = _sc_aggregate2(p2, src3, dst3, zrows8)

    out8 = pl.pallas_call(
        _tc_out_body,
        out_shape=jax.ShapeDtypeStruct((_N, 8), jnp.float32),
    )(agg2p[:_N], cntp[:_N, :1], r2)

    return out8[:, :2]


# speedup vs baseline: 1.0015x; 1.0015x over previous
import functools

import jax
import jax.numpy as jnp
from jax import lax
from jax.experimental import pallas as pl
from jax.experimental.pallas import tpu as pltpu
from jax.experimental.pallas import tpu_sc as plsc

_N = 10000
_E = 320000
_D_IN = 128
_D_H = 64

_NC = 2
_NS = 16
_CHUNK = 128
_CPT = 160
_EP = _NS * _CPT * _CHUNK
_R = 10112
_RPT = _R // _NS
_DEPTH = 4


def _sc_mesh():
    return plsc.VectorSubcoreMesh(core_axis_name="c", subcore_axis_name="s",
                                  num_cores=_NC, num_subcores=_NS)


def _sc_agg_body(with_count, *refs):
    if with_count:
        (tbl, src3, dst3, zrows, zrows8, ones, aggp, cntp,
         agg_sh, cnt_sh, sidx2, didx2, ones_v, csem,
         *bufs) = refs
    else:
        (tbl, src3, dst3, zrows, aggp,
         agg_sh, sidx2, didx2, *bufs) = refs
    rows = bufs[:_DEPTH]
    gsems = bufs[_DEPTH:2 * _DEPTH]
    ssems = bufs[2 * _DEPTH:]
    c = lax.axis_index("c")
    s = lax.axis_index("s")
    rbase = s * _RPT
    cpt = _CPT + c * 0

    @pl.when(c == 0)
    def _gather_core():
        pltpu.sync_copy(zrows, agg_sh.at[pl.ds(rbase, _RPT)])
        pltpu.sync_copy(src3.at[s], sidx2)
        pltpu.sync_copy(dst3.at[s], didx2)
        plsc.subcore_barrier()

        for m in range(_DEPTH - 1):
            pltpu.async_copy(tbl.at[sidx2.at[m]], rows[m], gsems[m])

        def step(j, carry):
            for m in range(_DEPTH):
                @pl.when(j % _DEPTH == m)
                def _(m=m):
                    pltpu.make_async_copy(tbl.at[sidx2.at[j]], rows[m],
                                          gsems[m]).wait()
                    pltpu.async_copy(rows[m], agg_sh.at[didx2.at[j]],
                                     ssems[m], add=True)
                    nxt = j + _DEPTH - 1
                    b = (m + _DEPTH - 1) % _DEPTH

                    @pl.when(nxt < cpt)
                    def _():
                        @pl.when(j >= 1)
                        def _():
                            pltpu.make_async_copy(rows[b],
                                                  agg_sh.at[didx2.at[0]],
                                                  ssems[b]).wait()

                        pltpu.async_copy(tbl.at[sidx2.at[nxt]], rows[b],
                                         gsems[b])
            return carry

        lax.fori_loop(0, cpt, step, 0)
        for m in range(_DEPTH):
            pltpu.make_async_copy(rows[m], agg_sh.at[didx2.at[0]],
                                  ssems[m]).wait()
        plsc.subcore_barrier()
        pltpu.sync_copy(agg_sh.at[pl.ds(rbase, _RPT)],
                        aggp.at[pl.ds(rbase, _RPT)])

    if with_count:
        @pl.when(c == 1)
        def _count_core():
            pltpu.sync_copy(zrows8, cnt_sh.at[pl.ds(rbase, _RPT)])
            pltpu.sync_copy(ones, ones_v)
            pltpu.sync_copy(dst3.at[s], didx2)
            plsc.subcore_barrier()

            def cstep(j, carry):
                pltpu.async_copy(ones_v, cnt_sh.at[didx2.at[j]], csem,
                                 add=True)

                @pl.when(j >= _DEPTH)
                def _():
                    pltpu.make_async_copy(ones_v, cnt_sh.at[didx2.at[0]],
                                          csem).wait()
                return carry

            lax.fori_loop(0, cpt, cstep, 0)
            for _ in range(_DEPTH):
                pltpu.make_async_copy(ones_v, cnt_sh.at[didx2.at[0]],
                                      csem).wait()
            plsc.subcore_barrier()
            pltpu.sync_copy(cnt_sh.at[pl.ds(rbase, _RPT)],
                            cntp.at[pl.ds(rbase, _RPT)])


def _sc_aggregate1(p1, src3, dst3, zrows, zrows8, ones):
    fn = pl.kernel(
        functools.partial(_sc_agg_body, True),
        out_type=(
            jax.ShapeDtypeStruct((_R, _D_H), jnp.float32),
            jax.ShapeDtypeStruct((_R, 8), jnp.float32),
        ),
        mesh=_sc_mesh(),
        compiler_params=pltpu.CompilerParams(use_tc_tiling_on_sc=False),
        scratch_types=[
            pltpu.VMEM_SHARED((_R, _D_H), jnp.float32),
            pltpu.VMEM_SHARED((_R, 8), jnp.float32),
            pltpu.VMEM((_CPT, _CHUNK), jnp.int32),
            pltpu.VMEM((_CPT, _CHUNK), jnp.int32),
            pltpu.VMEM((_CHUNK, 8), jnp.float32),
            pltpu.SemaphoreType.DMA,
        ] + [pltpu.VMEM((_CHUNK, _D_H), jnp.float32)] * _DEPTH
          + [pltpu.SemaphoreType.DMA] * (2 * _DEPTH),
    )
    return fn(p1, src3, dst3, zrows, zrows8, ones)


def _sc_aggregate2(p2, src3, dst3, zrows8):
    fn = pl.kernel(
        functools.partial(_sc_agg_body, False),
        out_type=jax.ShapeDtypeStruct((_R, 8), jnp.float32),
        mesh=_sc_mesh(),
        compiler_params=pltpu.CompilerParams(use_tc_tiling_on_sc=False),
        scratch_types=[
            pltpu.VMEM_SHARED((_R, 8), jnp.float32),
            pltpu.VMEM((_CPT, _CHUNK), jnp.int32),
            pltpu.VMEM((_CPT, _CHUNK), jnp.int32),
        ] + [pltpu.VMEM((_CHUNK, 8), jnp.float32)] * _DEPTH
          + [pltpu.SemaphoreType.DMA] * (2 * _DEPTH),
    )
    return fn(p2, src3, dst3, zrows8)


def _tc_proj1_body(x_ref, wl_ref, wr_ref, b_ref, p_ref, r_ref):
    x = x_ref[...]
    p_ref[...] = jnp.dot(x, wl_ref[...], preferred_element_type=jnp.float32)
    r_ref[...] = (jnp.dot(x, wr_ref[...], preferred_element_type=jnp.float32)
                  + b_ref[...])


def _tc_mid_body(a_ref, c_ref, r1_ref, wl_ref, wr_ref,
                 b_ref, p2_ref, r2_ref):
    cnt = jnp.maximum(c_ref[...], 1.0)
    h = jnp.maximum(a_ref[...] / cnt + r1_ref[...], 0.0)
    p2_ref[...] = jnp.dot(h, wl_ref[...], preferred_element_type=jnp.float32)
    r2_ref[...] = (jnp.dot(h, wr_ref[...], preferred_element_type=jnp.float32)
                   + b_ref[...])


def _tc_out_body(a_ref, c_ref, r2_ref, o_ref):
    cnt = jnp.maximum(c_ref[...], 1.0)
    o_ref[...] = a_ref[...] / cnt + r2_ref[...]


def kernel(x, edge_index, W1_l, W1_r, b1, W2_l, W2_r, b2):
    src = edge_index[0]
    dst = edge_index[1]
    pad = _EP - _E
    src3 = jnp.concatenate([src, jnp.zeros((pad,), jnp.int32)]
                           ).reshape(_NS, _CPT, _CHUNK)
    dst3 = jnp.concatenate([dst, jnp.full((pad,), _N, jnp.int32)]
                           ).reshape(_NS, _CPT, _CHUNK)
    zrows = jnp.zeros((_RPT, _D_H), jnp.float32)
    zrows8 = jnp.zeros((_RPT, 8), jnp.float32)
    ones = jnp.ones((_CHUNK, 8), jnp.float32)

    p1, r1 = pl.pallas_call(
        _tc_proj1_body,
        out_shape=(
            jax.ShapeDtypeStruct((_N, _D_H), jnp.float32),
            jax.ShapeDtypeStruct((_N, _D_H), jnp.float32),
        ),
    )(x, W1_l.T, W1_r.T, b1[None, :])

    aggp, cntp = _sc_aggregate1(p1, src3, dst3, zrows, zrows8, ones)

    w2l8 = jnp.zeros((_D_H, 8), jnp.float32).at[:, :2].set(W2_l.T)
    w2r8 = jnp.zeros((_D_H, 8), jnp.float32).at[:, :2].set(W2_r.T)
    b2_8 = jnp.zeros((1, 8), jnp.float32).at[0, :2].set(b2)
    p2, r2 = pl.pallas_call(
        _tc_mid_body,
        out_shape=(
            jax.ShapeDtypeStruct((_N, 8), jnp.float32),
            jax.ShapeDtypeStruct((_N, 8), jnp.float32),
        ),
    )(aggp[:_N], cntp[:_N, :1], r1, w2l8, w2r8, b2_8)

    agg2p = _sc_aggregate2(p2, src3, dst3, zrows8)

    out8 = pl.pallas_call(
        _tc_out_body,
        out_shape=jax.ShapeDtypeStruct((_N, 8), jnp.float32),
    )(agg2p[:_N], cntp[:_N, :1], r2)

    return out8[:, :2]


# speedup vs baseline: 1.8552x; 1.8523x over previous
import functools

import jax
import jax.numpy as jnp
from jax import lax
from jax.experimental import pallas as pl
from jax.experimental.pallas import tpu as pltpu
from jax.experimental.pallas import tpu_sc as plsc

_N = 10000
_E = 320000
_D_IN = 128
_D_H = 64

_NC = 2
_NS = 16
_CHUNK = 128
_CPT = 160
_EP = _NS * _CPT * _CHUNK
_R = 10112
_RPT = _R // _NS
_DEPTH = 4


def _sc_mesh():
    return plsc.VectorSubcoreMesh(core_axis_name="c", subcore_axis_name="s",
                                  num_cores=_NC, num_subcores=_NS)


def _sc_agg_body(with_count, *refs):
    if with_count:
        (tbl, src3, dst3, zrows, zrows8, ones, aggp, cntp,
         agg_sh, cnt_sh, sidx2, didx2, ones_v, csem,
         *bufs) = refs
    else:
        (tbl, src3, dst3, zrows, aggp,
         agg_sh, sidx2, didx2, *bufs) = refs
    rows = bufs[:_DEPTH]
    gsems = bufs[_DEPTH:2 * _DEPTH]
    ssems = bufs[2 * _DEPTH:]
    c = lax.axis_index("c")
    s = lax.axis_index("s")
    rbase = s * _RPT
    cpt = _CPT + c * 0

    @pl.when(c == 0)
    def _gather_core():
        pltpu.sync_copy(zrows, agg_sh.at[pl.ds(rbase, _RPT)])
        pltpu.sync_copy(src3.at[s], sidx2)
        pltpu.sync_copy(dst3.at[s], didx2)
        plsc.subcore_barrier()

        for m in range(_DEPTH - 1):
            pltpu.async_copy(tbl.at[sidx2.at[m]], rows[m], gsems[m])

        def step(j, carry):
            for m in range(_DEPTH):
                @pl.when(j % _DEPTH == m)
                def _(m=m):
                    pltpu.make_async_copy(tbl.at[sidx2.at[j]], rows[m],
                                          gsems[m]).wait()
                    pltpu.async_copy(rows[m], agg_sh.at[didx2.at[j]],
                                     ssems[m], add=True)
                    nxt = j + _DEPTH - 1
                    b = (m + _DEPTH - 1) % _DEPTH

                    @pl.when(nxt < cpt)
                    def _():
                        @pl.when(j >= 1)
                        def _():
                            pltpu.make_async_copy(rows[b],
                                                  agg_sh.at[didx2.at[0]],
                                                  ssems[b]).wait()

                        pltpu.async_copy(tbl.at[sidx2.at[nxt]], rows[b],
                                         gsems[b])
            return carry

        lax.fori_loop(0, cpt, step, 0)
        for m in range(_DEPTH):
            pltpu.make_async_copy(rows[m], agg_sh.at[didx2.at[0]],
                                  ssems[m]).wait()
        plsc.subcore_barrier()
        pltpu.sync_copy(agg_sh.at[pl.ds(rbase, _RPT)],
                        aggp.at[pl.ds(rbase, _RPT)])

    if with_count:
        @pl.when(c == 1)
        def _count_core():
            pltpu.sync_copy(zrows8, cnt_sh.at[pl.ds(rbase, _RPT)])
            pltpu.sync_copy(ones, ones_v)
            pltpu.sync_copy(dst3.at[s], didx2)
            plsc.subcore_barrier()

            def cstep(j, carry):
                pltpu.async_copy(ones_v, cnt_sh.at[didx2.at[j]], csem,
                                 add=True)

                @pl.when(j >= _DEPTH)
                def _():
                    pltpu.make_async_copy(ones_v, cnt_sh.at[didx2.at[0]],
                                          csem).wait()
                return carry

            lax.fori_loop(0, cpt, cstep, 0)
            for _ in range(_DEPTH):
                pltpu.make_async_copy(ones_v, cnt_sh.at[didx2.at[0]],
                                      csem).wait()
            plsc.subcore_barrier()
            pltpu.sync_copy(cnt_sh.at[pl.ds(rbase, _RPT)],
                            cntp.at[pl.ds(rbase, _RPT)])


def _sc_aggregate1(p1, src3, dst3, zrows, zrows8, ones):
    fn = pl.kernel(
        functools.partial(_sc_agg_body, True),
        out_type=(
            jax.ShapeDtypeStruct((_R, _D_H), jnp.float32),
            jax.ShapeDtypeStruct((_R, 8), jnp.float32),
        ),
        mesh=_sc_mesh(),
        compiler_params=pltpu.CompilerParams(use_tc_tiling_on_sc=False),
        scratch_types=[
            pltpu.VMEM_SHARED((_R, _D_H), jnp.float32),
            pltpu.VMEM_SHARED((_R, 8), jnp.float32),
            pltpu.VMEM((_CPT, _CHUNK), jnp.int32),
            pltpu.VMEM((_CPT, _CHUNK), jnp.int32),
            pltpu.VMEM((_CHUNK, 8), jnp.float32),
            pltpu.SemaphoreType.DMA,
        ] + [pltpu.VMEM((_CHUNK, _D_H), jnp.float32)] * _DEPTH
          + [pltpu.SemaphoreType.DMA] * (2 * _DEPTH),
    )
    return fn(p1, src3, dst3, zrows, zrows8, ones)


def _sc_aggregate2(p2, src3, dst3, zrows8):
    fn = pl.kernel(
        functools.partial(_sc_agg_body, False),
        out_type=jax.ShapeDtypeStruct((_R, 8), jnp.float32),
        mesh=_sc_mesh(),
        compiler_params=pltpu.CompilerParams(use_tc_tiling_on_sc=False),
        scratch_types=[
            pltpu.VMEM_SHARED((_R, 8), jnp.float32),
            pltpu.VMEM((_CPT, _CHUNK), jnp.int32),
            pltpu.VMEM((_CPT, _CHUNK), jnp.int32),
        ] + [pltpu.VMEM((_CHUNK, 8), jnp.float32)] * _DEPTH
          + [pltpu.SemaphoreType.DMA] * (2 * _DEPTH),
    )
    return fn(p2, src3, dst3, zrows8)


def _tc_proj1_body(x_ref, wl_ref, wr_ref, b_ref, p_ref, r_ref):
    x = x_ref[...]
    p_ref[...] = jnp.dot(x, wl_ref[...], preferred_element_type=jnp.float32)
    r_ref[...] = (jnp.dot(x, wr_ref[...], preferred_element_type=jnp.float32)
                  + b_ref[...])


def _tc_mid_body(a_ref, c_ref, r1_ref, wl_ref, wr_ref,
                 b_ref, p2_ref, r2_ref):
    cnt = jnp.maximum(c_ref[...], 1.0)
    h = jnp.maximum(a_ref[...] / cnt + r1_ref[...], 0.0)
    p2_ref[...] = jnp.dot(h, wl_ref[...], preferred_element_type=jnp.float32)
    r2_ref[...] = (jnp.dot(h, wr_ref[...], preferred_element_type=jnp.float32)
                   + b_ref[...])


def _tc_out_body(a_ref, c_ref, r2_ref, o_ref):
    cnt = jnp.maximum(c_ref[...], 1.0)
    o_ref[...] = a_ref[...] / cnt + r2_ref[...]


def kernel(x, edge_index, W1_l, W1_r, b1, W2_l, W2_r, b2):
    src = edge_index[0]
    dst = edge_index[1]
    pad = _EP - _E
    pad_src = (jnp.arange(pad, dtype=jnp.int32) * 37) % _N
    pad_dst = _N + (jnp.arange(pad, dtype=jnp.int32) % (_R - _N))
    src3 = jnp.concatenate([src, pad_src]).reshape(_NS, _CPT, _CHUNK)
    dst3 = jnp.concatenate([dst, pad_dst]).reshape(_NS, _CPT, _CHUNK)
    zrows = jnp.zeros((_RPT, _D_H), jnp.float32)
    zrows8 = jnp.zeros((_RPT, 8), jnp.float32)
    ones = jnp.ones((_CHUNK, 8), jnp.float32)

    p1, r1 = pl.pallas_call(
        _tc_proj1_body,
        out_shape=(
            jax.ShapeDtypeStruct((_N, _D_H), jnp.float32),
            jax.ShapeDtypeStruct((_N, _D_H), jnp.float32),
        ),
    )(x, W1_l.T, W1_r.T, b1[None, :])

    aggp, cntp = _sc_aggregate1(p1, src3, dst3, zrows, zrows8, ones)

    w2l8 = jnp.zeros((_D_H, 8), jnp.float32).at[:, :2].set(W2_l.T)
    w2r8 = jnp.zeros((_D_H, 8), jnp.float32).at[:, :2].set(W2_r.T)
    b2_8 = jnp.zeros((1, 8), jnp.float32).at[0, :2].set(b2)
    p2, r2 = pl.pallas_call(
        _tc_mid_body,
        out_shape=(
            jax.ShapeDtypeStruct((_N, 8), jnp.float32),
            jax.ShapeDtypeStruct((_N, 8), jnp.float32),
        ),
    )(aggp[:_N], cntp[:_N, :1], r1, w2l8, w2r8, b2_8)

    agg2p = _sc_aggregate2(p2, src3, dst3, zrows8)

    out8 = pl.pallas_call(
        _tc_out_body,
        out_shape=jax.ShapeDtypeStruct((_N, 8), jnp.float32),
    )(agg2p[:_N], cntp[:_N, :1], r2)

    return out8[:, :2]


# speedup vs baseline: 2.2014x; 1.1867x over previous
import functools

import jax
import jax.numpy as jnp
from jax import lax
from jax.experimental import pallas as pl
from jax.experimental.pallas import tpu as pltpu
from jax.experimental.pallas import tpu_sc as plsc

_N = 10000
_E = 320000
_D_IN = 128
_D_H = 64

_NC = 2
_NS = 16
_NW = _NC * _NS
_CHUNK = 128
_TOT_CH = _E // _CHUNK
_CPT_LO = _TOT_CH // _NW
_HI_W = _TOT_CH - _CPT_LO * _NW
_CPT_HI = _CPT_LO + 1
_R = 10112
_RPT = _R // _NS
_DEPTH = 4


def _sc_mesh():
    return plsc.VectorSubcoreMesh(core_axis_name="c", subcore_axis_name="s",
                                  num_cores=_NC, num_subcores=_NS)


def _sc_agg_body(with_count, *refs):
    if with_count:
        (tbl, src2, dst2, zrows, zrows8, ones, aggp, cntp,
         agg_sh, cnt_sh, sidx2, didx2, ones_v, csem,
         *bufs) = refs
    else:
        (tbl, src2, dst2, zrows, aggp,
         agg_sh, sidx2, didx2, *bufs) = refs
    rows = bufs[:_DEPTH]
    gsems = bufs[_DEPTH:2 * _DEPTH]
    ssems = bufs[2 * _DEPTH:]
    c = lax.axis_index("c")
    s = lax.axis_index("s")
    w = c * _NS + s
    rbase = s * _RPT
    count = jnp.where(w < _HI_W, _CPT_HI, _CPT_LO)
    base = w * _CPT_LO + jnp.minimum(w, _HI_W)

    pltpu.sync_copy(zrows, agg_sh.at[pl.ds(rbase, _RPT)])
    if with_count:
        pltpu.sync_copy(zrows8, cnt_sh.at[pl.ds(rbase, _RPT)])
        pltpu.sync_copy(ones, ones_v)

    @pl.when(w < _HI_W)
    def _():
        pltpu.sync_copy(src2.at[pl.ds(base, _CPT_HI)],
                        sidx2.at[pl.ds(0, _CPT_HI)])
        pltpu.sync_copy(dst2.at[pl.ds(base, _CPT_HI)],
                        didx2.at[pl.ds(0, _CPT_HI)])

    @pl.when(w >= _HI_W)
    def _():
        pltpu.sync_copy(src2.at[pl.ds(base, _CPT_LO)],
                        sidx2.at[pl.ds(0, _CPT_LO)])
        pltpu.sync_copy(dst2.at[pl.ds(base, _CPT_LO)],
                        didx2.at[pl.ds(0, _CPT_LO)])

    plsc.subcore_barrier()

    for m in range(_DEPTH - 1):
        pltpu.async_copy(tbl.at[sidx2.at[m]], rows[m], gsems[m])

    def step(j, carry):
        for m in range(_DEPTH):
            @pl.when(j % _DEPTH == m)
            def _(m=m):
                pltpu.make_async_copy(tbl.at[sidx2.at[j]], rows[m],
                                      gsems[m]).wait()
                pltpu.async_copy(rows[m], agg_sh.at[didx2.at[j]], ssems[m],
                                 add=True)
                if with_count:
                    pltpu.async_copy(ones_v, cnt_sh.at[didx2.at[j]], csem,
                                     add=True)

                    @pl.when(j >= _DEPTH)
                    def _():
                        pltpu.make_async_copy(ones_v,
                                              cnt_sh.at[didx2.at[0]],
                                              csem).wait()
                nxt = j + _DEPTH - 1
                b = (m + _DEPTH - 1) % _DEPTH

                @pl.when(nxt < count)
                def _():
                    @pl.when(j >= 1)
                    def _():
                        pltpu.make_async_copy(rows[b],
                                              agg_sh.at[didx2.at[0]],
                                              ssems[b]).wait()

                    pltpu.async_copy(tbl.at[sidx2.at[nxt]], rows[b],
                                     gsems[b])
        return carry

    lax.fori_loop(0, count, step, 0)

    for m in range(_DEPTH):
        pltpu.make_async_copy(rows[m], agg_sh.at[didx2.at[0]],
                              ssems[m]).wait()
    if with_count:
        for _ in range(_DEPTH):
            pltpu.make_async_copy(ones_v, cnt_sh.at[didx2.at[0]],
                                  csem).wait()
    plsc.subcore_barrier()

    pltpu.sync_copy(agg_sh.at[pl.ds(rbase, _RPT)],
                    aggp.at[c, pl.ds(rbase, _RPT)])
    if with_count:
        pltpu.sync_copy(cnt_sh.at[pl.ds(rbase, _RPT)],
                        cntp.at[c, pl.ds(rbase, _RPT)])


def _sc_aggregate1(p1, src2, dst2, zrows, zrows8, ones):
    fn = pl.kernel(
        functools.partial(_sc_agg_body, True),
        out_type=(
            jax.ShapeDtypeStruct((_NC, _R, _D_H), jnp.float32),
            jax.ShapeDtypeStruct((_NC, _R, 8), jnp.float32),
        ),
        mesh=_sc_mesh(),
        compiler_params=pltpu.CompilerParams(use_tc_tiling_on_sc=False),
        scratch_types=[
            pltpu.VMEM_SHARED((_R, _D_H), jnp.float32),
            pltpu.VMEM_SHARED((_R, 8), jnp.float32),
            pltpu.VMEM((_CPT_HI, _CHUNK), jnp.int32),
            pltpu.VMEM((_CPT_HI, _CHUNK), jnp.int32),
            pltpu.VMEM((_CHUNK, 8), jnp.float32),
            pltpu.SemaphoreType.DMA,
        ] + [pltpu.VMEM((_CHUNK, _D_H), jnp.float32)] * _DEPTH
          + [pltpu.SemaphoreType.DMA] * (2 * _DEPTH),
    )
    return fn(p1, src2, dst2, zrows, zrows8, ones)


def _sc_aggregate2(p2, src2, dst2, zrows8):
    fn = pl.kernel(
        functools.partial(_sc_agg_body, False),
        out_type=jax.ShapeDtypeStruct((_NC, _R, 8), jnp.float32),
        mesh=_sc_mesh(),
        compiler_params=pltpu.CompilerParams(use_tc_tiling_on_sc=False),
        scratch_types=[
            pltpu.VMEM_SHARED((_R, 8), jnp.float32),
            pltpu.VMEM((_CPT_HI, _CHUNK), jnp.int32),
            pltpu.VMEM((_CPT_HI, _CHUNK), jnp.int32),
        ] + [pltpu.VMEM((_CHUNK, 8), jnp.float32)] * _DEPTH
          + [pltpu.SemaphoreType.DMA] * (2 * _DEPTH),
    )
    return fn(p2, src2, dst2, zrows8)


def _tc_proj1_body(x_ref, wl_ref, wr_ref, b_ref, p_ref, r_ref):
    x = x_ref[...]
    p_ref[...] = jnp.dot(x, wl_ref[...], preferred_element_type=jnp.float32)
    r_ref[...] = (jnp.dot(x, wr_ref[...], preferred_element_type=jnp.float32)
                  + b_ref[...])


def _tc_mid_body(a_ref, c_ref, r1_ref, wl_ref, wr_ref,
                 b_ref, p2_ref, r2_ref):
    cnt = jnp.maximum(c_ref[0, :, :1] + c_ref[1, :, :1], 1.0)
    h = jnp.maximum((a_ref[0] + a_ref[1]) / cnt + r1_ref[...], 0.0)
    p2_ref[...] = jnp.dot(h, wl_ref[...], preferred_element_type=jnp.float32)
    r2_ref[...] = (jnp.dot(h, wr_ref[...], preferred_element_type=jnp.float32)
                   + b_ref[...])


def _tc_out_body(a_ref, c_ref, r2_ref, o_ref):
    cnt = jnp.maximum(c_ref[0, :, :1] + c_ref[1, :, :1], 1.0)
    o_ref[...] = (a_ref[0] + a_ref[1]) / cnt + r2_ref[...]


def kernel(x, edge_index, W1_l, W1_r, b1, W2_l, W2_r, b2):
    src2 = edge_index[0].reshape(_TOT_CH, _CHUNK)
    dst2 = edge_index[1].reshape(_TOT_CH, _CHUNK)
    xp = jnp.zeros((_R, _D_IN), jnp.float32).at[:_N].set(x)
    zrows = jnp.zeros((_RPT, _D_H), jnp.float32)
    zrows8 = jnp.zeros((_RPT, 8), jnp.float32)
    ones = jnp.ones((_CHUNK, 8), jnp.float32)

    p1, r1 = pl.pallas_call(
        _tc_proj1_body,
        out_shape=(
            jax.ShapeDtypeStruct((_R, _D_H), jnp.float32),
            jax.ShapeDtypeStruct((_R, _D_H), jnp.float32),
        ),
    )(xp, W1_l.T, W1_r.T, b1[None, :])

    aggp, cntp = _sc_aggregate1(p1, src2, dst2, zrows, zrows8, ones)

    w2l8 = jnp.zeros((_D_H, 8), jnp.float32).at[:, :2].set(W2_l.T)
    w2r8 = jnp.zeros((_D_H, 8), jnp.float32).at[:, :2].set(W2_r.T)
    b2_8 = jnp.zeros((1, 8), jnp.float32).at[0, :2].set(b2)
    p2, r2 = pl.pallas_call(
        _tc_mid_body,
        out_shape=(
            jax.ShapeDtypeStruct((_R, 8), jnp.float32),
            jax.ShapeDtypeStruct((_R, 8), jnp.float32),
        ),
    )(aggp, cntp, r1, w2l8, w2r8, b2_8)

    agg2p = _sc_aggregate2(p2, src2, dst2, zrows8)

    out8 = pl.pallas_call(
        _tc_out_body,
        out_shape=jax.ShapeDtypeStruct((_R, 8), jnp.float32),
    )(agg2p, cntp, r2)

    return out8[:_N, :2]
